# baseline (speedup 1.0000x reference)
import functools

import jax
import jax.numpy as jnp
from jax import lax
from jax.experimental import pallas as pl
from jax.experimental.pallas import tpu as pltpu
from jax.experimental.pallas import tpu_sc as plsc

N_LIG = 10000
N_PROT = 10000
E = 320000
D = 128
H = 128

NC = 2
NS = 16
NW = NC * NS
CHUNK = 128
CPT = 80
E_PAD = NW * CPT * CHUNK
NCHUNK = E_PAD // CHUNK
CPT_C = (48, 112)
CPTMAX = max(CPT_C)
ROWS = 10240
RPT = ROWS // NS
DUMMY = N_LIG
NB = 1
CW = 128

_NOTILE = pltpu.CompilerParams(use_tc_tiling_on_sc=False)


def _seg_sum_sc(xp_bf, idx_flat, rowids, zrows):
    mesh = plsc.VectorSubcoreMesh(core_axis_name="c", subcore_axis_name="s")

    @functools.partial(
        pl.kernel,
        out_type=(
            jax.ShapeDtypeStruct((NC, ROWS, D), jnp.float32),
        ),
        mesh=mesh,
        compiler_params=_NOTILE,
        scratch_types=[
            pltpu.VMEM((2, CPTMAX), jnp.int32),
            pltpu.VMEM((2, CPTMAX, CHUNK), jnp.int32),
            pltpu.VMEM((NB, CHUNK, D), jnp.float32),
            pltpu.VMEM_SHARED((ROWS, D), jnp.float32),
            pltpu.SemaphoreType.DMA,
        ],
    )
    def seg(xp_hbm, idxf_hbm, rid_hbm, z_hbm, psum_hbm,
            rid_v, idx_v, rows_v, accum_sh, sem):
        c = lax.axis_index("c")
        s = lax.axis_index("s")
        wid = s * NC + c

        pltpu.sync_copy(rid_hbm.at[wid], rid_v)
        pltpu.async_copy(idxf_hbm.at[rid_v.at[0]], idx_v.at[0], sem).wait()
        pltpu.async_copy(idxf_hbm.at[rid_v.at[1]], idx_v.at[1], sem).wait()

        def zslab(k, _):
            pltpu.sync_copy(z_hbm,
                            accum_sh.at[pl.ds(s * RPT + k * CHUNK, CHUNK)])
            return _
        lax.fori_loop(0, RPT // CHUNK, zslab, None)
        plsc.subcore_barrier()

        def body(i, _):
            j0 = NB * i
            for b in range(NB):
                pltpu.async_copy(xp_hbm.at[idx_v.at[0, j0 + b]],
                                 rows_v.at[b], sem)
            for b in range(NB):
                pltpu.make_async_copy(xp_hbm.at[idx_v.at[0, 0]],
                                      rows_v.at[b], sem).wait()
            for b in range(NB):
                pltpu.sync_copy(rows_v.at[b],
                                accum_sh.at[idx_v.at[1, j0 + b]], add=True)
            return _
        nblk = jnp.where(c == 0, CPT_C[0] // NB, CPT_C[1] // NB)
        lax.fori_loop(0, nblk, body, None)
        plsc.subcore_barrier()

        pltpu.sync_copy(accum_sh.at[pl.ds(s * RPT, RPT)],
                        psum_hbm.at[c, pl.ds(s * RPT, RPT)])

    return seg(xp_bf, idx_flat, rowids, zrows)


def _seg_cnt_sc(idx_flat, rowids_dst, ones_rows, zcnt):
    mesh = plsc.VectorSubcoreMesh(core_axis_name="c", subcore_axis_name="s")

    @functools.partial(
        pl.kernel,
        out_type=(
            jax.ShapeDtypeStruct((NC, ROWS, CW), jnp.float32),
        ),
        mesh=mesh,
        scratch_types=[
            pltpu.VMEM((CPT,), jnp.int32),
            pltpu.VMEM((CPT, CHUNK), jnp.int32),
            pltpu.VMEM((CHUNK, CW), jnp.float32),
            pltpu.VMEM_SHARED((ROWS, CW), jnp.float32),
            pltpu.SemaphoreType.DMA,
        ],
    )
    def cntk(idxf_hbm, rid_hbm, ones_hbm, z_hbm, pcnt_hbm,
             rid_v, dst_v, ones_v, cnt_sh, sem):
        c = lax.axis_index("c")
        s = lax.axis_index("s")
        wid = s * NC + c

        pltpu.sync_copy(rid_hbm.at[wid], rid_v)
        pltpu.async_copy(idxf_hbm.at[rid_v], dst_v, sem).wait()
        pltpu.sync_copy(ones_hbm, ones_v)

        def zslab(k, _):
            pltpu.sync_copy(z_hbm,
                            cnt_sh.at[pl.ds(s * RPT + k * CHUNK, CHUNK)])
            return _
        lax.fori_loop(0, RPT // CHUNK, zslab, None)
        plsc.subcore_barrier()

        def body(j, _):
            pltpu.sync_copy(ones_v, cnt_sh.at[dst_v.at[j]], add=True)
            return _
        lax.fori_loop(0, CPT, body, None)
        plsc.subcore_barrier()

        pltpu.sync_copy(cnt_sh.at[pl.ds(s * RPT, RPT)],
                        pcnt_hbm.at[c, pl.ds(s * RPT, RPT)])

    return cntk(idx_flat, rowids_dst, ones_rows, zcnt)


def _tail_tc(psum, pcnt, xl_pad, W_l, b_l, W_r, W_lin, b_lin):
    def body(ps_ref, pc_ref, xl_ref, wl_ref, bl_ref, wr_ref, wlin_ref,
             blin_ref, out_ref):
        ssum = (ps_ref[0].astype(jnp.float32)
                + ps_ref[1].astype(jnp.float32))
        cnt = (pc_ref[0, :, 0:1].astype(jnp.float32)
               + pc_ref[1, :, 0:1].astype(jnp.float32))
        mean = ssum / jnp.maximum(cnt, 1.0)
        z = (jnp.dot(mean, wl_ref[...], preferred_element_type=jnp.float32)
             + bl_ref[...]
             + jnp.dot(xl_ref[...], wr_ref[...],
                       preferred_element_type=jnp.float32))
        h = jnp.maximum(z, 0.0)
        rid = lax.broadcasted_iota(jnp.int32, (ROWS, 1), 0)
        h = jnp.where(rid < N_LIG, h, 0.0)
        m = jnp.sum(h, axis=0, keepdims=True) * (1.0 / N_LIG)
        out_ref[...] = (jnp.dot(m, wlin_ref[...],
                                preferred_element_type=jnp.float32)
                        + blin_ref[...])

    out = pl.pallas_call(
        body,
        out_shape=jax.ShapeDtypeStruct((1, 1), jnp.float32),
    )(psum, pcnt, xl_pad, W_l, b_l.reshape(1, H), W_r, W_lin,
      b_lin.reshape(1, 1))
    return out.reshape(1)


def kernel(x_ligand, x_protein, edge_index_lp, edge_index_pl,
           W_lp_l, b_lp_l, W_lp_r, W_pl_l, b_pl_l, W_pl_r, W_lin, b_lin):
    src = edge_index_pl[0].astype(jnp.int32)
    dst = edge_index_pl[1].astype(jnp.int32)
    pad = E_PAD - E
    src_t = jnp.concatenate([src, jnp.zeros((pad,), jnp.int32)]).reshape(
        NW, CPT, CHUNK)
    dst_t = jnp.concatenate([dst, jnp.full((pad,), DUMMY, jnp.int32)]).reshape(
        NW, CPT, CHUNK)

    zrows = jnp.zeros((CHUNK, D), jnp.float32)
    ones_rows = jnp.ones((CHUNK, CW), jnp.float32)
    zcnt = jnp.zeros((CHUNK, CW), jnp.float32)
    idx_flat = jnp.concatenate([src_t.reshape(NCHUNK, CHUNK),
                                dst_t.reshape(NCHUNK, CHUNK)], axis=0)
    ws = jnp.arange(NW, dtype=jnp.int32)
    s_ = ws // NC
    c_ = ws % NC
    kk = jnp.arange(CPTMAX, dtype=jnp.int32)
    cid = jnp.where(
        (c_[:, None] == 0),
        s_[:, None] * CPT_C[0] + jnp.minimum(kk[None, :], CPT_C[0] - 1),
        NS * CPT_C[0] + s_[:, None] * CPT_C[1]
        + jnp.minimum(kk[None, :], CPT_C[1] - 1))
    rowids = jnp.stack([cid, NCHUNK + cid], axis=1)
    rowids_dst = (NCHUNK + ws[:, None] * CPT
                  + jnp.arange(CPT, dtype=jnp.int32)[None, :])

    (psum,) = _seg_sum_sc(x_protein, idx_flat, rowids, zrows)
    (pcnt,) = _seg_cnt_sc(idx_flat, rowids_dst, ones_rows, zcnt)

    xl_pad = jnp.zeros((ROWS, D), jnp.float32).at[:N_LIG].set(x_ligand)
    return _tail_tc(psum, pcnt, xl_pad, W_pl_l, b_pl_l, W_pl_r, W_lin, b_lin)


# speedup vs baseline: 1.1962x; 1.1962x over previous
import functools

import jax
import jax.numpy as jnp
from jax import lax
from jax.experimental import pallas as pl
from jax.experimental.pallas import tpu as pltpu
from jax.experimental.pallas import tpu_sc as plsc

N_LIG = 10000
N_PROT = 10000
E = 320000
D = 128
H = 128

NC = 2
NS = 16
NW = NC * NS
CHUNK = 128
CPT = 80
E_PAD = NW * CPT * CHUNK
NCHUNK = E_PAD // CHUNK
CPT_C = (112, 48)
CPTMAX = max(CPT_C)
ROWS = 10240
RPT = ROWS // NS
DUMMY = N_LIG
NB = 1
CW = 128

_NOTILE = pltpu.CompilerParams(use_tc_tiling_on_sc=False)


def _seg_sum_sc(xp_bf, idx_flat, rowids, zrows):
    mesh = plsc.VectorSubcoreMesh(core_axis_name="c", subcore_axis_name="s")

    @functools.partial(
        pl.kernel,
        out_type=(
            jax.ShapeDtypeStruct((NC, ROWS, D), jnp.float32),
        ),
        mesh=mesh,
        compiler_params=_NOTILE,
        scratch_types=[
            pltpu.VMEM((2, CPTMAX), jnp.int32),
            pltpu.VMEM((2, CPTMAX, CHUNK), jnp.int32),
            pltpu.VMEM((NB, CHUNK, D), jnp.float32),
            pltpu.VMEM_SHARED((ROWS, D), jnp.float32),
            pltpu.SemaphoreType.DMA,
        ],
    )
    def seg(xp_hbm, idxf_hbm, rid_hbm, z_hbm, psum_hbm,
            rid_v, idx_v, rows_v, accum_sh, sem):
        c = lax.axis_index("c")
        s = lax.axis_index("s")
        wid = s * NC + c

        pltpu.sync_copy(rid_hbm.at[wid], rid_v)
        pltpu.async_copy(idxf_hbm.at[rid_v.at[0]], idx_v.at[0], sem).wait()
        pltpu.async_copy(idxf_hbm.at[rid_v.at[1]], idx_v.at[1], sem).wait()

        def zslab(k, _):
            pltpu.sync_copy(z_hbm,
                            accum_sh.at[pl.ds(s * RPT + k * CHUNK, CHUNK)])
            return _
        lax.fori_loop(0, RPT // CHUNK, zslab, None)
        plsc.subcore_barrier()

        def body(i, _):
            j0 = NB * i
            for b in range(NB):
                pltpu.async_copy(xp_hbm.at[idx_v.at[0, j0 + b]],
                                 rows_v.at[b], sem)
            for b in range(NB):
                pltpu.make_async_copy(xp_hbm.at[idx_v.at[0, 0]],
                                      rows_v.at[b], sem).wait()
            for b in range(NB):
                pltpu.sync_copy(rows_v.at[b],
                                accum_sh.at[idx_v.at[1, j0 + b]], add=True)
            return _
        nblk = jnp.where(c == 0, CPT_C[0] // NB, CPT_C[1] // NB)
        lax.fori_loop(0, nblk, body, None)
        plsc.subcore_barrier()

        pltpu.sync_copy(accum_sh.at[pl.ds(s * RPT, RPT)],
                        psum_hbm.at[c, pl.ds(s * RPT, RPT)])

    return seg(xp_bf, idx_flat, rowids, zrows)


def _seg_cnt_sc(idx_flat, rowids_dst, ones_rows, zcnt):
    mesh = plsc.VectorSubcoreMesh(core_axis_name="c", subcore_axis_name="s")

    @functools.partial(
        pl.kernel,
        out_type=(
            jax.ShapeDtypeStruct((NC, ROWS, CW), jnp.float32),
        ),
        mesh=mesh,
        scratch_types=[
            pltpu.VMEM((CPT,), jnp.int32),
            pltpu.VMEM((CPT, CHUNK), jnp.int32),
            pltpu.VMEM((CHUNK, CW), jnp.float32),
            pltpu.VMEM_SHARED((ROWS, CW), jnp.float32),
            pltpu.SemaphoreType.DMA,
        ],
    )
    def cntk(idxf_hbm, rid_hbm, ones_hbm, z_hbm, pcnt_hbm,
             rid_v, dst_v, ones_v, cnt_sh, sem):
        c = lax.axis_index("c")
        s = lax.axis_index("s")
        wid = s * NC + c

        pltpu.sync_copy(rid_hbm.at[wid], rid_v)
        pltpu.async_copy(idxf_hbm.at[rid_v], dst_v, sem).wait()
        pltpu.sync_copy(ones_hbm, ones_v)

        def zslab(k, _):
            pltpu.sync_copy(z_hbm,
                            cnt_sh.at[pl.ds(s * RPT + k * CHUNK, CHUNK)])
            return _
        lax.fori_loop(0, RPT // CHUNK, zslab, None)
        plsc.subcore_barrier()

        def body(j, _):
            pltpu.sync_copy(ones_v, cnt_sh.at[dst_v.at[j]], add=True)
            return _
        lax.fori_loop(0, CPT, body, None)
        plsc.subcore_barrier()

        pltpu.sync_copy(cnt_sh.at[pl.ds(s * RPT, RPT)],
                        pcnt_hbm.at[c, pl.ds(s * RPT, RPT)])

    return cntk(idx_flat, rowids_dst, ones_rows, zcnt)


def _tail_tc(psum, pcnt, xl_pad, W_l, b_l, W_r, W_lin, b_lin):
    def body(ps_ref, pc_ref, xl_ref, wl_ref, bl_ref, wr_ref, wlin_ref,
             blin_ref, out_ref):
        ssum = (ps_ref[0].astype(jnp.float32)
                + ps_ref[1].astype(jnp.float32))
        cnt = (pc_ref[0, :, 0:1].astype(jnp.float32)
               + pc_ref[1, :, 0:1].astype(jnp.float32))
        mean = ssum / jnp.maximum(cnt, 1.0)
        z = (jnp.dot(mean, wl_ref[...], preferred_element_type=jnp.float32)
             + bl_ref[...]
             + jnp.dot(xl_ref[...], wr_ref[...],
                       preferred_element_type=jnp.float32))
        h = jnp.maximum(z, 0.0)
        rid = lax.broadcasted_iota(jnp.int32, (ROWS, 1), 0)
        h = jnp.where(rid < N_LIG, h, 0.0)
        m = jnp.sum(h, axis=0, keepdims=True) * (1.0 / N_LIG)
        out_ref[...] = (jnp.dot(m, wlin_ref[...],
                                preferred_element_type=jnp.float32)
                        + blin_ref[...])

    out = pl.pallas_call(
        body,
        out_shape=jax.ShapeDtypeStruct((1, 1), jnp.float32),
    )(psum, pcnt, xl_pad, W_l, b_l.reshape(1, H), W_r, W_lin,
      b_lin.reshape(1, 1))
    return out.reshape(1)


def kernel(x_ligand, x_protein, edge_index_lp, edge_index_pl,
           W_lp_l, b_lp_l, W_lp_r, W_pl_l, b_pl_l, W_pl_r, W_lin, b_lin):
    src = edge_index_pl[0].astype(jnp.int32)
    dst = edge_index_pl[1].astype(jnp.int32)
    pad = E_PAD - E
    src_t = jnp.concatenate([src, jnp.zeros((pad,), jnp.int32)]).reshape(
        NW, CPT, CHUNK)
    dst_t = jnp.concatenate([dst, jnp.full((pad,), DUMMY, jnp.int32)]).reshape(
        NW, CPT, CHUNK)

    zrows = jnp.zeros((CHUNK, D), jnp.float32)
    ones_rows = jnp.ones((CHUNK, CW), jnp.float32)
    zcnt = jnp.zeros((CHUNK, CW), jnp.float32)
    idx_flat = jnp.concatenate([src_t.reshape(NCHUNK, CHUNK),
                                dst_t.reshape(NCHUNK, CHUNK)], axis=0)
    ws = jnp.arange(NW, dtype=jnp.int32)
    s_ = ws // NC
    c_ = ws % NC
    kk = jnp.arange(CPTMAX, dtype=jnp.int32)
    cid = jnp.where(
        (c_[:, None] == 0),
        s_[:, None] * CPT_C[0] + jnp.minimum(kk[None, :], CPT_C[0] - 1),
        NS * CPT_C[0] + s_[:, None] * CPT_C[1]
        + jnp.minimum(kk[None, :], CPT_C[1] - 1))
    rowids = jnp.stack([cid, NCHUNK + cid], axis=1)
    rowids_dst = (NCHUNK + ws[:, None] * CPT
                  + jnp.arange(CPT, dtype=jnp.int32)[None, :])

    (psum,) = _seg_sum_sc(x_protein, idx_flat, rowids, zrows)
    (pcnt,) = _seg_cnt_sc(idx_flat, rowids_dst, ones_rows, zcnt)

    xl_pad = jnp.zeros((ROWS, D), jnp.float32).at[:N_LIG].set(x_ligand)
    return _tail_tc(psum, pcnt, xl_pad, W_pl_l, b_pl_l, W_pl_r, W_lin, b_lin)
